# baseline (speedup 1.0000x reference)
import functools

import jax
import jax.numpy as jnp
from jax import lax
from jax.experimental import pallas as pl
from jax.experimental.pallas import tpu as pltpu
from jax.experimental.pallas import tpu_sc as plsc

_A = 26
_N = 100000
_B = 4096
_NC = 2
_NS = 16
_L = 16
_ROWS = _B // 128


_LSE_CHUNK = 12800
_LSE_GRID = -(-_N // _LSE_CHUNK)


def _lse_body(x_ref, out_ref, m_ref, s_ref):
    k = pl.program_id(0)

    @pl.when(k == 0)
    def _():
        m_ref[...] = jnp.full((_A, 1), -jnp.inf, jnp.float32)
        s_ref[...] = jnp.zeros((_A, 1), jnp.float32)

    x = x_ref[...]
    col = k * _LSE_CHUNK + jax.lax.broadcasted_iota(jnp.int32, x.shape, 1)
    x = jnp.where(col < _N, x, -jnp.inf)
    bm = jnp.max(x, axis=1, keepdims=True)
    m_old = m_ref[...]
    m_new = jnp.maximum(m_old, bm)
    s_ref[...] = s_ref[...] * jnp.exp(m_old - m_new) + jnp.sum(
        jnp.exp(x - m_new), axis=1, keepdims=True
    )
    m_ref[...] = m_new

    @pl.when(k == _LSE_GRID - 1)
    def _():
        out_ref[0, 0] = jnp.sum(m_new + jnp.log(s_ref[...]))


def _lse_sum(logits):
    return pl.pallas_call(
        _lse_body,
        grid=(_LSE_GRID,),
        in_specs=[pl.BlockSpec((_A, _LSE_CHUNK), lambda k: (0, k))],
        out_shape=jax.ShapeDtypeStruct((1, 1), jnp.float32),
        out_specs=pl.BlockSpec(memory_space=pltpu.SMEM),
        scratch_shapes=[
            pltpu.VMEM((_A, 1), jnp.float32),
            pltpu.VMEM((_A, 1), jnp.float32),
        ],
    )(logits)


@functools.lru_cache(maxsize=1)
def _make_gather_kernel():
    mesh = plsc.VectorSubcoreMesh(core_axis_name="c", subcore_axis_name="s")

    @functools.partial(
        pl.kernel,
        mesh=mesh,
        compiler_params=pltpu.CompilerParams(needs_layout_passes=False),
        out_type=[
            jax.ShapeDtypeStruct((_ROWS, 128), jnp.float32),
            jax.ShapeDtypeStruct((_ROWS, 128), jnp.float32),
        ],
        scratch_types=[
            pltpu.VMEM((_N,), jnp.float32),
            pltpu.VMEM((_B,), jnp.int32),
            pltpu.VMEM((_ROWS, 128), jnp.float32),
            pltpu.VMEM((_ROWS,), jnp.int32),
            pltpu.VMEM_SHARED((_ROWS, 128), jnp.float32),
            pltpu.SemaphoreType.DMA,
        ],
    )
    def k(logits_hbm, vt_hbm, out_a, out_b, row_v, idx_v, part_v, sidx_v, shared,
          sem):
        cid = lax.axis_index("c")
        sid = lax.axis_index("s")
        row = cid * 13 + sid
        active = sid < 13

        sidx_v[pl.ds(0, _L)] = lax.iota(jnp.int32, _L)
        sidx_v[pl.ds(_L, _L)] = lax.iota(jnp.int32, _L) + _L

        @pl.when(active)
        def _():
            cp = pltpu.async_copy(logits_hbm.at[row], row_v, sem)
            pltpu.sync_copy(vt_hbm.at[row], idx_v)
            cp.wait()

            @plsc.parallel_loop(0, _ROWS, step=1, unroll=2)
            def _(r):
                for j in range(8):
                    idx16 = idx_v[pl.ds(r * 128 + j * _L, _L)]
                    g = plsc.load_gather(row_v, [idx16])
                    part_v[r, pl.ds(j * _L, _L)] = g

        @pl.when(sid == 0)
        def _():
            pltpu.sync_copy(part_v, shared)

        plsc.subcore_barrier()

        @pl.when(jnp.logical_and(active, sid != 0))
        def _():
            pltpu.sync_copy(part_v, shared.at[sidx_v], add=True)

        plsc.subcore_barrier()

        @pl.when(jnp.logical_and(sid == 0, cid == 0))
        def _():
            pltpu.sync_copy(shared, out_a)

        @pl.when(jnp.logical_and(sid == 0, cid == 1))
        def _():
            pltpu.sync_copy(shared, out_b)

    return k


def kernel(logits, value):
    pa, pb = _make_gather_kernel()(logits, value.T)
    c = _lse_sum(logits)[0, 0]
    return (pa + pb - c).reshape(_B)


# speedup vs baseline: 1.0222x; 1.0222x over previous
import functools

import jax
import jax.numpy as jnp
from jax import lax
from jax.experimental import pallas as pl
from jax.experimental.pallas import tpu as pltpu
from jax.experimental.pallas import tpu_sc as plsc

_A = 26
_N = 100000
_B = 4096
_NC = 2
_NS = 16
_L = 16
_ROWS = _B // 128


_LSE_CHUNK = 51200
_LSE_GRID = 2


def _lse_body(x_ref, out_ref, m_ref, s_ref):
    k = pl.program_id(0)

    @pl.when(k == 0)
    def _():
        m_ref[...] = jnp.full((_A, 1), -jnp.inf, jnp.float32)
        s_ref[...] = jnp.zeros((_A, 1), jnp.float32)

    x = x_ref[...]
    col = k * _LSE_CHUNK + jax.lax.broadcasted_iota(jnp.int32, x.shape, 1)
    x = jnp.where(col < _N, x, -jnp.inf)
    bm = jnp.max(x, axis=1, keepdims=True)
    m_old = m_ref[...]
    m_new = jnp.maximum(m_old, bm)
    s_ref[...] = s_ref[...] * jnp.exp(m_old - m_new) + jnp.sum(
        jnp.exp(x - m_new), axis=1, keepdims=True
    )
    m_ref[...] = m_new

    @pl.when(k == _LSE_GRID - 1)
    def _():
        out_ref[0, 0] = jnp.sum(m_new + jnp.log(s_ref[...]))


def _lse_sum(logits):
    return pl.pallas_call(
        _lse_body,
        grid=(_LSE_GRID,),
        in_specs=[pl.BlockSpec((_A, _LSE_CHUNK), lambda k: (0, k))],
        out_shape=jax.ShapeDtypeStruct((1, 1), jnp.float32),
        out_specs=pl.BlockSpec(memory_space=pltpu.SMEM),
        scratch_shapes=[
            pltpu.VMEM((_A, 1), jnp.float32),
            pltpu.VMEM((_A, 1), jnp.float32),
        ],
    )(logits)


@functools.lru_cache(maxsize=1)
def _make_gather_kernel():
    mesh = plsc.VectorSubcoreMesh(core_axis_name="c", subcore_axis_name="s")

    @functools.partial(
        pl.kernel,
        mesh=mesh,
        compiler_params=pltpu.CompilerParams(needs_layout_passes=False),
        out_type=[
            jax.ShapeDtypeStruct((_ROWS, 128), jnp.float32),
            jax.ShapeDtypeStruct((_ROWS, 128), jnp.float32),
        ],
        scratch_types=[
            pltpu.VMEM((_N,), jnp.float32),
            pltpu.VMEM((_B,), jnp.int32),
            pltpu.VMEM((_ROWS, 128), jnp.float32),
            pltpu.VMEM((_ROWS,), jnp.int32),
            pltpu.VMEM_SHARED((_ROWS, 128), jnp.float32),
            pltpu.SemaphoreType.DMA,
        ],
    )
    def k(logits_hbm, vt_hbm, out_a, out_b, row_v, idx_v, part_v, sidx_v, shared,
          sem):
        cid = lax.axis_index("c")
        sid = lax.axis_index("s")
        row = cid * 13 + sid
        active = sid < 13

        sidx_v[pl.ds(0, _L)] = lax.iota(jnp.int32, _L)
        sidx_v[pl.ds(_L, _L)] = lax.iota(jnp.int32, _L) + _L

        @pl.when(active)
        def _():
            cp = pltpu.async_copy(logits_hbm.at[row], row_v, sem)
            pltpu.sync_copy(vt_hbm.at[row], idx_v)
            cp.wait()

            @plsc.parallel_loop(0, _ROWS, step=1, unroll=2)
            def _(r):
                for j in range(8):
                    idx16 = idx_v[pl.ds(r * 128 + j * _L, _L)]
                    g = plsc.load_gather(row_v, [idx16])
                    part_v[r, pl.ds(j * _L, _L)] = g

        @pl.when(sid == 0)
        def _():
            pltpu.sync_copy(part_v, shared)

        plsc.subcore_barrier()

        @pl.when(jnp.logical_and(active, sid != 0))
        def _():
            pltpu.sync_copy(part_v, shared.at[sidx_v], add=True)

        plsc.subcore_barrier()

        @pl.when(jnp.logical_and(sid == 0, cid == 0))
        def _():
            pltpu.sync_copy(shared, out_a)

        @pl.when(jnp.logical_and(sid == 0, cid == 1))
        def _():
            pltpu.sync_copy(shared, out_b)

    return k


def kernel(logits, value):
    pa, pb = _make_gather_kernel()(logits, value.T)
    c = _lse_sum(logits)[0, 0]
    return (pa + pb - c).reshape(_B)
